# baseline (speedup 1.0000x reference)
import functools

import jax
import jax.numpy as jnp
from jax import lax
from jax.experimental import pallas as pl
from jax.experimental.pallas import tpu as pltpu
from jax.experimental.pallas import tpu_sc as plsc

N = 10000
E = 320000
D = 128
EPS = 1e-5

NC = 2
NS = 16
NW = NC * NS
EPW = E // NW
NP = 10240
RPT = NP // NS

DCH = 80
DNCH = EPW // DCH
DZR = 32

MCH = 25
MNCH = EPW // MCH
BLK = 8
NBLK = MNCH // BLK
NWIN = 12
UB = 4
MZR = 16


def _fire_drain_rows(src_of, dst_of, n, sem):
    def fire(i, carry):
        pltpu.async_copy(src_of(i), dst_of(i), sem)
        return carry

    lax.fori_loop(0, n, fire, 0)

    def drain(i, carry):
        pltpu.make_async_copy(src_of(i), dst_of(i), sem).wait()
        return carry

    lax.fori_loop(0, n, drain, 0)


@functools.cache
def _get_deg_kernel():
    mesh = plsc.VectorSubcoreMesh(core_axis_name="c", subcore_axis_name="s")
    return functools.partial(
        pl.kernel,
        mesh=mesh,
        out_type=jax.ShapeDtypeStruct((NC, NP, 16), jnp.float32),
        scratch_types=[
            pltpu.VMEM((DNCH, DCH), jnp.int32),
            pltpu.VMEM((DCH, 16), jnp.float32),
            pltpu.VMEM((DZR, 16), jnp.float32),
            pltpu.SemaphoreType.DMA,
            pltpu.SemaphoreType.DMA,
            pltpu.VMEM_SHARED((NP, 16), jnp.float32),
        ],
    )(_deg_body)


def _deg_body(dst_hbm, out_hbm, idx_v, ones_v, zero_v, sem, zsem, acc):
    c = lax.axis_index("c")
    s = lax.axis_index("s")
    wid = s * NC + c

    idx_cp = pltpu.async_copy(dst_hbm.at[wid], idx_v, sem)

    def fill(i, carry):
        ones_v[i, :] = jnp.ones((16,), jnp.float32)
        return carry

    lax.fori_loop(0, DCH, fill, 0)

    def zfill(i, carry):
        zero_v[i, :] = jnp.zeros((16,), jnp.float32)
        return carry

    lax.fori_loop(0, DZR, zfill, 0)

    _fire_drain_rows(lambda i: zero_v,
                     lambda i: acc.at[pl.ds(s * RPT + i * DZR, DZR)],
                     RPT // DZR, zsem)
    idx_cp.wait()
    plsc.subcore_barrier()

    U = 5

    def step(i, carry):
        handles = []
        for k in range(U):
            handles.append(pltpu.async_copy(
                ones_v, acc.at[idx_v.at[i * U + k]], zsem, add=True))
        for h in handles:
            h.wait()
        return carry

    lax.fori_loop(0, DNCH // U, step, 0)
    plsc.subcore_barrier()

    _fire_drain_rows(lambda i: acc.at[pl.ds(s * RPT + i * DZR, DZR)],
                     lambda i: out_hbm.at[c, pl.ds(s * RPT + i * DZR, DZR)],
                     RPT // DZR, zsem)


@functools.cache
def _get_msg_kernel():
    mesh = plsc.VectorSubcoreMesh(core_axis_name="c", subcore_axis_name="s")
    return functools.partial(
        pl.kernel,
        mesh=mesh,
        out_type=jax.ShapeDtypeStruct((NC, NP, D), jnp.float32),
        scratch_types=[
            pltpu.VMEM((4, BLK, MCH), jnp.int32),
            pltpu.VMEM((4, BLK, MCH), jnp.int32),
            pltpu.VMEM((UB, MCH, D), jnp.float32),
            pltpu.VMEM((MZR, D), jnp.float32),
            [pltpu.SemaphoreType.DMA] * 4,
            [pltpu.SemaphoreType.DMA] * 4,
            [pltpu.SemaphoreType.DMA] * 4,
            pltpu.SemaphoreType.DMA,
            pltpu.VMEM_SHARED((NP, D), jnp.float32),
        ],
    )(_msg_body)


def _msg_body(hs_hbm, src_hbm, dst_hbm, out_hbm, idxs_v, idxd_v, rows_v,
              zero_v, gsems, ssems, isems, zsem, acc):
    c = lax.axis_index("c")
    s = lax.axis_index("s")
    wid = s * NC + c

    def load_idx(blk, slot):
        boff = blk * BLK
        pltpu.async_copy(src_hbm.at[wid, pl.ds(boff, BLK)],
                         idxs_v.at[slot], isems[slot])
        pltpu.async_copy(dst_hbm.at[wid, pl.ds(boff, BLK)],
                         idxd_v.at[slot], isems[slot])

    def drain_idx(blk, slot):
        boff = blk * BLK
        pltpu.make_async_copy(src_hbm.at[wid, pl.ds(boff, BLK)],
                              idxs_v.at[slot], isems[slot]).wait()
        pltpu.make_async_copy(dst_hbm.at[wid, pl.ds(boff, BLK)],
                              idxd_v.at[slot], isems[slot]).wait()

    def g_issue(slot, row, buf):
        pltpu.async_copy(hs_hbm.at[idxs_v.at[slot, row]], rows_v.at[buf],
                         gsems[buf])

    def g_drain(slot, row, buf):
        pltpu.make_async_copy(hs_hbm.at[idxs_v.at[slot, row]], rows_v.at[buf],
                              gsems[buf]).wait()

    def s_issue(slot, row, buf):
        pltpu.async_copy(rows_v.at[buf], acc.at[idxd_v.at[slot, row]],
                         ssems[buf], add=True)

    def s_drain(slot, row, buf):
        pltpu.make_async_copy(rows_v.at[buf], acc.at[idxd_v.at[slot, row]],
                              ssems[buf]).wait()

    def pos(cj):
        return (cj // BLK) % 4, cj % BLK, cj % UB

    load_idx(0, 0)
    load_idx(1, 1)

    def zfill(i, carry):
        j = i // 8
        k = i % 8
        zero_v[j, pl.ds(k * 16, 16)] = jnp.zeros((16,), jnp.float32)
        return carry

    lax.fori_loop(0, MZR * 8, zfill, 0)

    _fire_drain_rows(lambda i: zero_v,
                     lambda i: acc.at[pl.ds(s * RPT + i * MZR, MZR)],
                     RPT // MZR, zsem)
    drain_idx(0, 0)
    g_issue(0, 0, 0)
    g_issue(0, 1, 1)
    plsc.subcore_barrier()

    def window(t, carry):
        tb = t * 4

        for cj in range(32):
            sl, r, b = pos(cj)
            kk = cj // BLK
            jj = cj % BLK

            if jj == 1:
                load_idx(tb + kk + 2, (kk + 2) % 4)

            g_drain(sl, r, b)
            s_issue(sl, r, b)

            sl2, r2, b2 = pos((cj - 2) % 32)
            if cj < 2:
                @pl.when(t > 0)
                def _ds():
                    s_drain(sl2, r2, b2)
            else:
                s_drain(sl2, r2, b2)

            if jj == 6:
                drain_idx(tb + kk + 1, (kk + 1) % 4)

            sl3, r3, b3 = pos((cj + 2) % 32)
            g_issue(sl3, r3, b3)

        return carry

    lax.fori_loop(0, NWIN, window, 0)

    for cj in range(32, 48):
        sl, r, b = pos(cj % 32)
        kk = (cj % 32) // BLK
        jj = cj % BLK

        g_drain(sl, r, b)
        s_issue(sl, r, b)
        sl2, r2, b2 = pos((cj - 2) % 32)
        s_drain(sl2, r2, b2)
        if jj == 6 and kk == 0:
            drain_idx(NBLK - 1, 1)
        if cj < 46:
            sl3, r3, b3 = pos((cj + 2) % 32)
            g_issue(sl3, r3, b3)

    s_drain(*pos(46 % 32))
    s_drain(*pos(47 % 32))
    plsc.subcore_barrier()

    _fire_drain_rows(lambda i: acc.at[pl.ds(s * RPT + i * MZR, MZR)],
                     lambda i: out_hbm.at[c, pl.ds(s * RPT + i * MZR, MZR)],
                     RPT // MZR, zsem)


BN = 1000
GRID = N // BN


def _dinv_block(d0_ref, d1_ref):
    deg = d0_ref[...][:, 0] + d1_ref[...][:, 0] + 1.0
    return lax.rsqrt(deg)


def _mm1_body(x_ref, w_ref, d0_ref, d1_ref, o_ref):
    dinv = _dinv_block(d0_ref, d1_ref)
    h = jnp.dot(x_ref[...], w_ref[...], preferred_element_type=jnp.float32)
    o_ref[...] = h * dinv[:, None]


def _layer_body(a0_ref, a1_ref, hs_ref, d0_ref, d1_ref, w_ref, gs_ref, cb_ref,
                x_out_ref, hs_out_ref):
    dinv = _dinv_block(d0_ref, d1_ref)
    v = (a0_ref[...] + a1_ref[...] + hs_ref[...]) * dinv[:, None]
    x1 = jnp.maximum(v * gs_ref[...] + cb_ref[...], 0.0)
    x_out_ref[...] = x1
    h = jnp.dot(x1, w_ref[...], preferred_element_type=jnp.float32)
    hs_out_ref[...] = h * dinv[:, None]


def _final_body(a0_ref, a1_ref, hs_ref, d0_ref, d1_ref, gs_ref, cb_ref,
                x1_ref, lw1_ref, lb1_ref, lw2_ref, lb2_ref, o_ref):
    dinv = _dinv_block(d0_ref, d1_ref)
    v = (a0_ref[...] + a1_ref[...] + hs_ref[...]) * dinv[:, None]
    x2 = jnp.maximum(v * gs_ref[...] + cb_ref[...], 0.0)
    lw1 = lw1_ref[...]
    h = (jnp.dot(x1_ref[...], lw1[:D], preferred_element_type=jnp.float32)
         + jnp.dot(x2, lw1[D:], preferred_element_type=jnp.float32)
         + lb1_ref[...])
    h = jnp.maximum(h, 0.0)
    o_ref[...] = (jnp.dot(h, lw2_ref[...], preferred_element_type=jnp.float32)
                  + lb2_ref[...])


def _row_spec(cols):
    return pl.BlockSpec((BN, cols), lambda i: (i, 0))


def _full_spec(rows, cols):
    return pl.BlockSpec((rows, cols), lambda i: (0, 0))


def _mm1(x, w1, d0, d1):
    return pl.pallas_call(
        _mm1_body,
        grid=(GRID,),
        in_specs=[_row_spec(D), _full_spec(D, D), _row_spec(16), _row_spec(16)],
        out_specs=_row_spec(D),
        out_shape=jax.ShapeDtypeStruct((N, D), jnp.float32),
    )(x, w1, d0, d1)


def _layer(a0, a1, hs, d0, d1, w, gs, cb):
    return pl.pallas_call(
        _layer_body,
        grid=(GRID,),
        in_specs=[_row_spec(D), _row_spec(D), _row_spec(D), _row_spec(16),
                  _row_spec(16), _full_spec(D, D), _full_spec(1, D),
                  _full_spec(1, D)],
        out_specs=[_row_spec(D), _row_spec(D)],
        out_shape=[jax.ShapeDtypeStruct((N, D), jnp.float32),
                   jax.ShapeDtypeStruct((N, D), jnp.float32)],
    )(a0, a1, hs, d0, d1, w, gs, cb)


def _final(a0, a1, hs, d0, d1, gs, cb, x1, lw1, lb1, lw2, lb2):
    return pl.pallas_call(
        _final_body,
        grid=(GRID,),
        in_specs=[_row_spec(D), _row_spec(D), _row_spec(D), _row_spec(16),
                  _row_spec(16), _full_spec(1, D), _full_spec(1, D),
                  _row_spec(D), _full_spec(2 * D, D), _full_spec(1, D),
                  _full_spec(D, D), _full_spec(1, D)],
        out_specs=_row_spec(D),
        out_shape=jax.ShapeDtypeStruct((N, D), jnp.float32),
    )(a0, a1, hs, d0, d1, gs, cb, x1, lw1, lb1, lw2, lb2)


def kernel(x, edge_index, W1, b1, g1, be1, W2, b2, g2, be2, LW1, Lb1, LW2, Lb2):
    bscale = lax.rsqrt(jnp.float32(1.0 + EPS))
    gs1 = (g1 * bscale)[None, :]
    cb1 = (b1 * gs1[0] + be1)[None, :]
    gs2 = (g2 * bscale)[None, :]
    cb2 = (b2 * gs2[0] + be2)[None, :]

    src = edge_index[0]
    dst = edge_index[1]
    dst_wd = dst.reshape(NW, DNCH, DCH)
    src_wm = src.reshape(NW, MNCH, MCH)
    dst_wm = dst.reshape(NW, MNCH, MCH)

    deg2 = _get_deg_kernel()(dst_wd)
    d0, d1 = deg2[0], deg2[1]

    hs1 = _mm1(x, W1, d0, d1)
    acc1 = _get_msg_kernel()(hs1, src_wm, dst_wm)
    x1, hs2 = _layer(acc1[0], acc1[1], hs1, d0, d1, W2, gs1, cb1)
    acc2 = _get_msg_kernel()(hs2, src_wm, dst_wm)
    out = _final(acc2[0], acc2[1], hs2, d0, d1, gs2, cb2, x1, LW1,
                 Lb1[None, :], LW2, Lb2[None, :])
    return out


# speedup vs baseline: 1.0126x; 1.0126x over previous
import functools

import jax
import jax.numpy as jnp
from jax import lax
from jax.experimental import pallas as pl
from jax.experimental.pallas import tpu as pltpu
from jax.experimental.pallas import tpu_sc as plsc

N = 10000
E = 320000
D = 128
EPS = 1e-5

NC = 2
NS = 16
NW = NC * NS
EPW = E // NW
NP = 10240
RPT = NP // NS

DCH = 80
DNCH = EPW // DCH
DZR = 32

MCH = 25
MNCH = EPW // MCH
BLK = 8
NBLK = MNCH // BLK
NWIN = 12
UB = 4
MZR = 16


def _fire_drain_rows(src_of, dst_of, n, sem):
    def fire(i, carry):
        pltpu.async_copy(src_of(i), dst_of(i), sem)
        return carry

    lax.fori_loop(0, n, fire, 0)

    def drain(i, carry):
        pltpu.make_async_copy(src_of(i), dst_of(i), sem).wait()
        return carry

    lax.fori_loop(0, n, drain, 0)


@functools.cache
def _get_deg_kernel():
    mesh = plsc.VectorSubcoreMesh(core_axis_name="c", subcore_axis_name="s")
    return functools.partial(
        pl.kernel,
        mesh=mesh,
        out_type=jax.ShapeDtypeStruct((NC, NP, 16), jnp.float32),
        scratch_types=[
            pltpu.VMEM((DNCH, DCH), jnp.int32),
            pltpu.VMEM((DCH, 16), jnp.float32),
            pltpu.VMEM((DZR, 16), jnp.float32),
            pltpu.SemaphoreType.DMA,
            pltpu.SemaphoreType.DMA,
            pltpu.VMEM_SHARED((NP, 16), jnp.float32),
        ],
    )(_deg_body)


def _deg_body(dst_hbm, out_hbm, idx_v, ones_v, zero_v, sem, zsem, acc):
    c = lax.axis_index("c")
    s = lax.axis_index("s")
    wid = s * NC + c

    idx_cp = pltpu.async_copy(dst_hbm.at[wid], idx_v, sem)

    def fill(i, carry):
        ones_v[i, :] = jnp.ones((16,), jnp.float32)
        return carry

    lax.fori_loop(0, DCH, fill, 0)

    def zfill(i, carry):
        zero_v[i, :] = jnp.zeros((16,), jnp.float32)
        return carry

    lax.fori_loop(0, DZR, zfill, 0)

    _fire_drain_rows(lambda i: zero_v,
                     lambda i: acc.at[pl.ds(s * RPT + i * DZR, DZR)],
                     RPT // DZR, zsem)
    idx_cp.wait()
    plsc.subcore_barrier()

    U = 5

    def step(i, carry):
        handles = []
        for k in range(U):
            handles.append(pltpu.async_copy(
                ones_v, acc.at[idx_v.at[i * U + k]], zsem, add=True))
        for h in handles:
            h.wait()
        return carry

    lax.fori_loop(0, DNCH // U, step, 0)
    plsc.subcore_barrier()

    _fire_drain_rows(lambda i: acc.at[pl.ds(s * RPT + i * DZR, DZR)],
                     lambda i: out_hbm.at[c, pl.ds(s * RPT + i * DZR, DZR)],
                     RPT // DZR, zsem)


@functools.cache
def _get_msg_kernel():
    mesh = plsc.VectorSubcoreMesh(core_axis_name="c", subcore_axis_name="s")
    return functools.partial(
        pl.kernel,
        mesh=mesh,
        out_type=jax.ShapeDtypeStruct((NC, NP, D), jnp.float32),
        scratch_types=[
            pltpu.VMEM((2, BLK, MCH), jnp.int32),
            pltpu.VMEM((2, BLK, MCH), jnp.int32),
            pltpu.VMEM((UB, MCH, D), jnp.float32),
            pltpu.VMEM((MZR, D), jnp.float32),
            pltpu.SemaphoreType.DMA,
            pltpu.SemaphoreType.DMA,
            pltpu.SemaphoreType.DMA,
            pltpu.SemaphoreType.DMA,
            pltpu.SemaphoreType.DMA,
            pltpu.SemaphoreType.DMA,
            pltpu.SemaphoreType.DMA,
            pltpu.VMEM_SHARED((NP, D), jnp.float32),
        ],
    )(_msg_body)


def _msg_body(hs_hbm, src_hbm, dst_hbm, out_hbm, idxs_v, idxd_v, rows_v,
              zero_v, g0, g1, g2, g3, ssem, is0, is1, acc):
    c = lax.axis_index("c")
    s = lax.axis_index("s")
    wid = s * NC + c
    gsems = (g0, g1, g2, g3)
    isems = (is0, is1)

    for sl in range(2):
        pltpu.async_copy(src_hbm.at[wid, pl.ds(sl * BLK, BLK)],
                         idxs_v.at[sl], isems[sl])
        pltpu.async_copy(dst_hbm.at[wid, pl.ds(sl * BLK, BLK)],
                         idxd_v.at[sl], isems[sl])

    def zfill(i, carry):
        j = i // 8
        k = i % 8
        zero_v[j, pl.ds(k * 16, 16)] = jnp.zeros((16,), jnp.float32)
        return carry

    lax.fori_loop(0, MZR * 8, zfill, 0)

    _fire_drain_rows(lambda i: zero_v,
                     lambda i: acc.at[pl.ds(s * RPT + i * MZR, MZR)],
                     RPT // MZR, ssem)
    plsc.subcore_barrier()

    NIT = NBLK // 2

    def it(t, carry):
        for sl in range(2):
            blk = 2 * t + sl
            boff = blk * BLK
            pltpu.make_async_copy(src_hbm.at[wid, pl.ds(boff, BLK)],
                                  idxs_v.at[sl], isems[sl]).wait()
            pltpu.make_async_copy(dst_hbm.at[wid, pl.ds(boff, BLK)],
                                  idxd_v.at[sl], isems[sl]).wait()
            for sp in range(2):
                gh = []
                for k in range(UB):
                    r = sp * UB + k
                    gh.append(pltpu.async_copy(
                        hs_hbm.at[idxs_v.at[sl, r]], rows_v.at[k], gsems[k]))
                sh = []
                for k in range(UB):
                    r = sp * UB + k
                    gh[k].wait()
                    sh.append(pltpu.async_copy(
                        rows_v.at[k], acc.at[idxd_v.at[sl, r]], ssem,
                        add=True))
                for h in sh:
                    h.wait()

            @pl.when(t < NIT - 1)
            def _prefetch():
                noff = (blk + 2) * BLK
                pltpu.async_copy(src_hbm.at[wid, pl.ds(noff, BLK)],
                                 idxs_v.at[sl], isems[sl])
                pltpu.async_copy(dst_hbm.at[wid, pl.ds(noff, BLK)],
                                 idxd_v.at[sl], isems[sl])

        return carry

    lax.fori_loop(0, NIT, it, 0)
    plsc.subcore_barrier()

    _fire_drain_rows(lambda i: acc.at[pl.ds(s * RPT + i * MZR, MZR)],
                     lambda i: out_hbm.at[c, pl.ds(s * RPT + i * MZR, MZR)],
                     RPT // MZR, ssem)


BN = 1000
GRID = N // BN


def _dinv_block(d0_ref, d1_ref):
    deg = d0_ref[...][:, 0] + d1_ref[...][:, 0] + 1.0
    return lax.rsqrt(deg)


def _mm1_body(x_ref, w_ref, d0_ref, d1_ref, o_ref):
    dinv = _dinv_block(d0_ref, d1_ref)
    h = jnp.dot(x_ref[...], w_ref[...], preferred_element_type=jnp.float32)
    o_ref[...] = h * dinv[:, None]


def _layer_body(a0_ref, a1_ref, hs_ref, d0_ref, d1_ref, w_ref, gs_ref, cb_ref,
                x_out_ref, hs_out_ref):
    dinv = _dinv_block(d0_ref, d1_ref)
    v = (a0_ref[...] + a1_ref[...] + hs_ref[...]) * dinv[:, None]
    x1 = jnp.maximum(v * gs_ref[...] + cb_ref[...], 0.0)
    x_out_ref[...] = x1
    h = jnp.dot(x1, w_ref[...], preferred_element_type=jnp.float32)
    hs_out_ref[...] = h * dinv[:, None]


def _final_body(a0_ref, a1_ref, hs_ref, d0_ref, d1_ref, gs_ref, cb_ref,
                x1_ref, lw1_ref, lb1_ref, lw2_ref, lb2_ref, o_ref):
    dinv = _dinv_block(d0_ref, d1_ref)
    v = (a0_ref[...] + a1_ref[...] + hs_ref[...]) * dinv[:, None]
    x2 = jnp.maximum(v * gs_ref[...] + cb_ref[...], 0.0)
    lw1 = lw1_ref[...]
    h = (jnp.dot(x1_ref[...], lw1[:D], preferred_element_type=jnp.float32)
         + jnp.dot(x2, lw1[D:], preferred_element_type=jnp.float32)
         + lb1_ref[...])
    h = jnp.maximum(h, 0.0)
    o_ref[...] = (jnp.dot(h, lw2_ref[...], preferred_element_type=jnp.float32)
                  + lb2_ref[...])


def _row_spec(cols):
    return pl.BlockSpec((BN, cols), lambda i: (i, 0))


def _full_spec(rows, cols):
    return pl.BlockSpec((rows, cols), lambda i: (0, 0))


def _mm1(x, w1, d0, d1):
    return pl.pallas_call(
        _mm1_body,
        grid=(GRID,),
        in_specs=[_row_spec(D), _full_spec(D, D), _row_spec(16), _row_spec(16)],
        out_specs=_row_spec(D),
        out_shape=jax.ShapeDtypeStruct((N, D), jnp.float32),
    )(x, w1, d0, d1)


def _layer(a0, a1, hs, d0, d1, w, gs, cb):
    return pl.pallas_call(
        _layer_body,
        grid=(GRID,),
        in_specs=[_row_spec(D), _row_spec(D), _row_spec(D), _row_spec(16),
                  _row_spec(16), _full_spec(D, D), _full_spec(1, D),
                  _full_spec(1, D)],
        out_specs=[_row_spec(D), _row_spec(D)],
        out_shape=[jax.ShapeDtypeStruct((N, D), jnp.float32),
                   jax.ShapeDtypeStruct((N, D), jnp.float32)],
    )(a0, a1, hs, d0, d1, w, gs, cb)


def _final(a0, a1, hs, d0, d1, gs, cb, x1, lw1, lb1, lw2, lb2):
    return pl.pallas_call(
        _final_body,
        grid=(GRID,),
        in_specs=[_row_spec(D), _row_spec(D), _row_spec(D), _row_spec(16),
                  _row_spec(16), _full_spec(1, D), _full_spec(1, D),
                  _row_spec(D), _full_spec(2 * D, D), _full_spec(1, D),
                  _full_spec(D, D), _full_spec(1, D)],
        out_specs=_row_spec(D),
        out_shape=jax.ShapeDtypeStruct((N, D), jnp.float32),
    )(a0, a1, hs, d0, d1, gs, cb, x1, lw1, lb1, lw2, lb2)


def kernel(x, edge_index, W1, b1, g1, be1, W2, b2, g2, be2, LW1, Lb1, LW2, Lb2):
    bscale = lax.rsqrt(jnp.float32(1.0 + EPS))
    gs1 = (g1 * bscale)[None, :]
    cb1 = (b1 * gs1[0] + be1)[None, :]
    gs2 = (g2 * bscale)[None, :]
    cb2 = (b2 * gs2[0] + be2)[None, :]

    src = edge_index[0]
    dst = edge_index[1]
    dst_wd = dst.reshape(NW, DNCH, DCH)
    src_wm = src.reshape(NW, MNCH, MCH)
    dst_wm = dst.reshape(NW, MNCH, MCH)

    deg2 = _get_deg_kernel()(dst_wd)
    d0, d1 = deg2[0], deg2[1]

    hs1 = _mm1(x, W1, d0, d1)
    acc1 = _get_msg_kernel()(hs1, src_wm, dst_wm)
    x1, hs2 = _layer(acc1[0], acc1[1], hs1, d0, d1, W2, gs1, cb1)
    acc2 = _get_msg_kernel()(hs2, src_wm, dst_wm)
    out = _final(acc2[0], acc2[1], hs2, d0, d1, gs2, cb2, x1, LW1,
                 Lb1[None, :], LW2, Lb2[None, :])
    return out
